# baseline (speedup 1.0000x reference)
import jax
import jax.numpy as jnp
from jax import lax
from jax.experimental import pallas as pl
from jax.experimental.pallas import tpu as pltpu
from jax.experimental.pallas import tpu_sc as plsc

N_ROWS = 100000
N_FEAT = 128
K_OUT = 2000

NS = 16
NC = 2
L = 16
C = 6272
P = NS * C
KPAD = 2048
G = KPAD // (NS * NC)
SEL = KPAD + NS * L
ZCH = SEL // NS


def _body(x_hbm, keys_hbm, out_hbm,
          keys_v, gt_ku, gt_idx, cura_ku, cura_idx, curb_ku, curb_idx,
          loc_ku, loc_idx, srt_ku, srt_idx, rnk_v, outidx_v, rows_v,
          cnt_l, cnt2_l, stage16, idx16, val16, tmp16, hist2,
          sh_cnt, sh_cnts2, sh_selku, sh_selidx, sh_outidx, sem):
  cid = lax.axis_index("c")
  sid = lax.axis_index("s")
  iota = lax.iota(jnp.int32, L)
  zeros = jnp.zeros((L,), jnp.int32)
  ones = jnp.ones((L,), jnp.int32)
  kvec = jnp.full((L,), K_OUT, jnp.int32)

  base = jnp.minimum(sid * C, jnp.int32(N_ROWS - C))
  pltpu.sync_copy(keys_hbm.at[pl.ds(base, C)], keys_v)

  ptr_gt = zeros
  gcount = zeros
  n_cur = jnp.full((L,), C, jnp.int32)

  for rnd in range(8):
    shift = jnp.uint32(28 - 4 * rnd)
    if rnd == 0:
      src_ku, src_idx = None, None
      dst_ku, dst_idx = cura_ku, cura_idx
    elif rnd % 2 == 1:
      src_ku, src_idx = cura_ku, cura_idx
      dst_ku, dst_idx = curb_ku, curb_idx
    else:
      src_ku, src_idx = curb_ku, curb_idx
      dst_ku, dst_idx = cura_ku, cura_idx

    def fetch(t):
      if rnd == 0:
        kf = keys_v[pl.ds(t * L, L)]
        b = lax.bitcast_convert_type(kf, jnp.int32)
        ku = jnp.where(b < 0, b ^ jnp.int32(0x7FFFFFFF), b)
        gidx = base + t * L + iota
        valid = gidx >= sid * C
      else:
        ku = src_ku[pl.ds(t * L, L)]
        gidx = src_idx[pl.ds(t * L, L)]
        valid = (t * L + iota) < n_cur
      ub = lax.bitcast_convert_type(ku, jnp.uint32) ^ jnp.uint32(0x80000000)
      dig = lax.bitcast_convert_type((ub >> shift) & jnp.uint32(15),
                                     jnp.int32)
      return ku, gidx, dig, valid

    nv = (jnp.max(n_cur) + L - 1) // L

    for q in range(16):
      hist2[pl.ds(q * L, L)] = zeros

    def hloop(t, _):
      _, _, dig, valid = fetch(t)
      plsc.addupdate_scatter(hist2, [dig * L + iota], ones, mask=valid)
      return 0
    lax.fori_loop(0, nv, hloop, 0)

    for j in range(16):
      s = plsc.cumsum(hist2[pl.ds(j * L, L)])
      plsc.store_scatter(stage16, [jnp.full((L,), j, jnp.int32)], s,
                         mask=(iota == L - 1))
    par = rnd % 2
    pltpu.sync_copy(stage16, sh_cnt.at[par, sid])
    plsc.subcore_barrier()
    pltpu.sync_copy(sh_cnt.at[par], cnt_l)
    ghist = cnt_l[0, :]
    for t in range(1, NS):
      ghist = ghist + cnt_l[t, :]
    suf = lax.rev(plsc.cumsum(lax.rev(ghist, (0,))), (0,))
    jstar = plsc.all_reduce_population_count((gcount + suf) >= kvec) - 1
    tmp16[...] = suf
    above = plsc.load_gather(tmp16, [jnp.minimum(jstar + 1, L - 1)])
    gcount = gcount + jnp.where(jstar >= L - 1, zeros, above)

    def ploop(t, carry):
      p_gt, p_nx = carry
      ku, gidx, dig, valid = fetch(t)
      mgt = (dig > jstar) & valid
      pos = p_gt + plsc.cumsum(jnp.where(mgt, ones, zeros)) - 1
      plsc.store_scatter(gt_ku, [pos], ku, mask=mgt)
      plsc.store_scatter(gt_idx, [pos], gidx, mask=mgt)
      meq = (dig == jstar) & valid
      pos2 = p_nx + plsc.cumsum(jnp.where(meq, ones, zeros)) - 1
      plsc.store_scatter(dst_ku, [pos2], ku, mask=meq)
      plsc.store_scatter(dst_idx, [pos2], gidx, mask=meq)
      return (p_gt + plsc.all_reduce_population_count(mgt),
              p_nx + plsc.all_reduce_population_count(meq))
    ptr_gt, n_cur = lax.fori_loop(0, nv, ploop, (ptr_gt, zeros))

  ptr_eq = n_cur
  eq_idx = curb_idx

  val16[...] = jnp.where(iota == 0, ptr_gt, jnp.where(iota == 1, ptr_eq, 0))
  idx16[...] = jnp.where(iota == 0, sid,
                         jnp.where(iota == 1, NS + sid, 2 * NS + iota))
  pltpu.sync_copy(val16, sh_cnts2.at[idx16])
  plsc.subcore_barrier()
  pltpu.sync_copy(sh_cnts2.at[pl.ds(0, 2 * NS)], cnt2_l)
  gtc = cnt2_l[pl.ds(0, L)]
  neqc = cnt2_l[pl.ds(L, L)]
  sid_v = jnp.full((L,), sid, jnp.int32)
  cg = plsc.cumsum(gtc)
  tmp16[...] = cg - gtc
  gt_base = plsc.load_gather(tmp16, [sid_v])
  tmp16[...] = cg
  ngt_tot = plsc.load_gather(tmp16, [jnp.full((L,), 15, jnp.int32)])
  tmp16[...] = plsc.cumsum(neqc) - neqc
  eq_base = plsc.load_gather(tmp16, [sid_v])
  take = jnp.clip(jnp.int32(K_OUT) - ngt_tot - eq_base, 0, ptr_eq)

  nev = (jnp.max(ptr_gt) + L - 1) // L

  def sort_one(t, _):
    i = t * L + iota
    ke = gt_ku[pl.ds(t * L, L)]
    ie = gt_idx[pl.ds(t * L, L)]

    def sinner(j, acc):
      jv = zeros + j
      kj = plsc.load_gather(gt_ku, [jv])
      ij = plsc.load_gather(gt_idx, [jv])
      c = (kj > ke) | ((kj == ke) & (ij < ie))
      return acc + jnp.where(c, ones, zeros)
    r = lax.fori_loop(0, jnp.max(ptr_gt), sinner, zeros)
    mvalid = i < ptr_gt
    plsc.store_scatter(srt_ku, [r], ke, mask=mvalid)
    plsc.store_scatter(srt_idx, [r], ie, mask=mvalid)
    return 0
  lax.fori_loop(0, nev, sort_one, 0)

  dump = KPAD + sid * L + iota

  def scat_gt(t, _):
    i = t * L + iota
    idx16[...] = jnp.where(i < ptr_gt, gt_base + i, dump)
    pltpu.sync_copy(srt_ku.at[pl.ds(t * L, L)], sh_selku.at[idx16])
    pltpu.sync_copy(srt_idx.at[pl.ds(t * L, L)], sh_selidx.at[idx16])
    return 0
  lax.fori_loop(0, nev, scat_gt, 0)

  nch2 = (jnp.max(take) + L - 1) // L

  def scat_eq(t, _):
    i = t * L + iota
    idx16[...] = jnp.where(i < take, ngt_tot + eq_base + i, dump)
    pltpu.sync_copy(eq_idx.at[pl.ds(t * L, L)], sh_outidx.at[idx16])
    return 0
  lax.fori_loop(0, nch2, scat_eq, 0)
  plsc.subcore_barrier()

  pltpu.sync_copy(sh_selku.at[pl.ds(0, KPAD)], loc_ku)
  pltpu.sync_copy(sh_selidx.at[pl.ds(0, KPAD)], loc_idx)
  stage16[...] = cg - gtc
  val16[...] = gtc
  maxv = zeros + jnp.max(gtc)
  nsteps = jnp.max(
      plsc.all_reduce_population_count((ones << iota) <= maxv))

  def search_one(ev, _):
    ke = srt_ku[pl.ds(ev * L, L)]
    ie = srt_idx[pl.ds(ev * L, L)]

    def tloop(t, rank):
      tv = zeros + t
      bt = plsc.load_gather(stage16, [tv])
      ct = plsc.load_gather(val16, [tv])

      def bstep(s, lohi):
        lo, hi = lohi
        act = lo < hi
        mid = (lo + hi) >> 1
        addr = bt + jnp.minimum(mid, jnp.maximum(ct - 1, zeros))
        kx = plsc.load_gather(loc_ku, [addr])
        ix = plsc.load_gather(loc_idx, [addr])
        g = (kx > ke) | ((kx == ke) & (ix < ie))
        return (jnp.where(act & g, mid + 1, lo),
                jnp.where(act & (~g), mid, hi))
      lo, _hi = lax.fori_loop(0, nsteps, bstep, (zeros, ct))
      return rank + lo
    rank = lax.fori_loop(0, NS, tloop, zeros)
    rnk_v[pl.ds(ev * L, L)] = rank
    return 0
  lax.fori_loop(0, nev, search_one, 0)

  def scat_rank(t, _):
    i = t * L + iota
    idx16[...] = jnp.where(i < ptr_gt, rnk_v[pl.ds(t * L, L)], dump)
    pltpu.sync_copy(srt_idx.at[pl.ds(t * L, L)], sh_outidx.at[idx16])
    return 0
  lax.fori_loop(0, nev, scat_rank, 0)
  plsc.subcore_barrier()

  wid = cid * NS + sid
  pltpu.sync_copy(sh_outidx.at[pl.ds(wid * G, G)], outidx_v)
  for q in range(G // L):
    v = outidx_v[pl.ds(q * L, L)]
    outidx_v[pl.ds(q * L, L)] = jnp.clip(v, 0, N_ROWS - 1)
  pltpu.async_copy(x_hbm.at[outidx_v], rows_v, sem).wait()
  pltpu.sync_copy(rows_v, out_hbm.at[pl.ds(wid * G, G)])


_sortpool = pl.kernel(
    _body,
    out_type=jax.ShapeDtypeStruct((KPAD, N_FEAT), jnp.float32),
    mesh=plsc.VectorSubcoreMesh(core_axis_name="c", subcore_axis_name="s"),
    compiler_params=pltpu.CompilerParams(needs_layout_passes=False),
    scratch_types=[
        pltpu.VMEM((C,), jnp.float32),
        pltpu.VMEM((KPAD,), jnp.int32),
        pltpu.VMEM((KPAD,), jnp.int32),
        pltpu.VMEM((C,), jnp.int32),
        pltpu.VMEM((C,), jnp.int32),
        pltpu.VMEM((C,), jnp.int32),
        pltpu.VMEM((C,), jnp.int32),
        pltpu.VMEM((KPAD,), jnp.int32),
        pltpu.VMEM((KPAD,), jnp.int32),
        pltpu.VMEM((KPAD,), jnp.int32),
        pltpu.VMEM((KPAD,), jnp.int32),
        pltpu.VMEM((KPAD,), jnp.int32),
        pltpu.VMEM((G,), jnp.int32),
        pltpu.VMEM((G, N_FEAT), jnp.float32),
        pltpu.VMEM((NS, L), jnp.int32),
        pltpu.VMEM((2 * NS,), jnp.int32),
        pltpu.VMEM((L,), jnp.int32),
        pltpu.VMEM((L,), jnp.int32),
        pltpu.VMEM((L,), jnp.int32),
        pltpu.VMEM((L,), jnp.int32),
        pltpu.VMEM((16 * L,), jnp.int32),
        pltpu.VMEM_SHARED((2, NS, L), jnp.int32),
        pltpu.VMEM_SHARED((3 * NS,), jnp.int32),
        pltpu.VMEM_SHARED((SEL,), jnp.int32),
        pltpu.VMEM_SHARED((SEL,), jnp.int32),
        pltpu.VMEM_SHARED((SEL,), jnp.int32),
        pltpu.SemaphoreType.DMA,
    ],
)


def kernel(X):
  out = _sortpool(X, X[:, N_FEAT - 1])
  return out[:K_OUT]


# speedup vs baseline: 1.0393x; 1.0393x over previous
import jax
import jax.numpy as jnp
from jax import lax
from jax.experimental import pallas as pl
from jax.experimental.pallas import tpu as pltpu
from jax.experimental.pallas import tpu_sc as plsc

N_ROWS = 100000
N_FEAT = 128
K_OUT = 2000

NS = 16
NC = 2
L = 16
C = 6272
P = NS * C
KPAD = 2048
G = KPAD // (NS * NC)
SEL = KPAD + NS * L
ZCH = SEL // NS


def _body(x_hbm, keys_hbm, out_hbm,
          keys_v, gt_ku, gt_idx, cura_ku, cura_idx, curb_ku, curb_idx,
          loc_ku, loc_idx, srt_ku, srt_idx, rnk_v, outidx_v, rows_v,
          cnt_l, cnt2_l, stage16, idx16, val16, tmp16, hist2,
          sh_cnt, sh_cnts2, sh_selku, sh_selidx, sh_outidx, sem):
  cid = lax.axis_index("c")
  sid = lax.axis_index("s")
  iota = lax.iota(jnp.int32, L)
  zeros = jnp.zeros((L,), jnp.int32)
  ones = jnp.ones((L,), jnp.int32)
  kvec = jnp.full((L,), K_OUT, jnp.int32)

  base = jnp.minimum(sid * C, jnp.int32(N_ROWS - C))
  pltpu.sync_copy(keys_hbm.at[pl.ds(base, C)], keys_v)

  ptr_gt = zeros
  gcount = zeros
  n_cur = jnp.full((L,), C, jnp.int32)

  for rnd in range(8):
    shift = jnp.uint32(28 - 4 * rnd)
    if rnd == 0:
      src_ku, src_idx = None, None
      dst_ku, dst_idx = cura_ku, cura_idx
    elif rnd % 2 == 1:
      src_ku, src_idx = cura_ku, cura_idx
      dst_ku, dst_idx = curb_ku, curb_idx
    else:
      src_ku, src_idx = curb_ku, curb_idx
      dst_ku, dst_idx = cura_ku, cura_idx

    def fetch(t):
      if rnd == 0:
        kf = keys_v[pl.ds(t * L, L)]
        b = lax.bitcast_convert_type(kf, jnp.int32)
        ku = jnp.where(b < 0, b ^ jnp.int32(0x7FFFFFFF), b)
        gidx = base + t * L + iota
        valid = gidx >= sid * C
      else:
        ku = src_ku[pl.ds(t * L, L)]
        gidx = src_idx[pl.ds(t * L, L)]
        valid = (t * L + iota) < n_cur
      ub = lax.bitcast_convert_type(ku, jnp.uint32) ^ jnp.uint32(0x80000000)
      dig = lax.bitcast_convert_type((ub >> shift) & jnp.uint32(15),
                                     jnp.int32)
      return ku, gidx, dig, valid

    nv = (jnp.max(n_cur) + L - 1) // L

    for q in range(16):
      hist2[pl.ds(q * L, L)] = zeros

    def hloop(t, _):
      _, _, dig, valid = fetch(t)
      plsc.addupdate_scatter(hist2, [dig * L + iota], ones, mask=valid)
      return 0
    lax.fori_loop(0, nv, hloop, 0)

    for j in range(16):
      s = plsc.cumsum(hist2[pl.ds(j * L, L)])
      plsc.store_scatter(stage16, [jnp.full((L,), j, jnp.int32)], s,
                         mask=(iota == L - 1))
    par = rnd % 2
    pltpu.sync_copy(stage16, sh_cnt.at[par, sid])
    plsc.subcore_barrier()
    pltpu.sync_copy(sh_cnt.at[par], cnt_l)
    ghist = cnt_l[0, :]
    for t in range(1, NS):
      ghist = ghist + cnt_l[t, :]
    suf = lax.rev(plsc.cumsum(lax.rev(ghist, (0,))), (0,))
    jstar = plsc.all_reduce_population_count((gcount + suf) >= kvec) - 1
    tmp16[...] = suf
    above = plsc.load_gather(tmp16, [jnp.minimum(jstar + 1, L - 1)])
    gcount = gcount + jnp.where(jstar >= L - 1, zeros, above)

    def ploop(t, carry):
      p_gt, p_nx = carry
      ku, gidx, dig, valid = fetch(t)
      mgt = (dig > jstar) & valid
      pos = p_gt + plsc.cumsum(jnp.where(mgt, ones, zeros)) - 1
      plsc.store_scatter(gt_ku, [pos], ku, mask=mgt)
      plsc.store_scatter(gt_idx, [pos], gidx, mask=mgt)
      meq = (dig == jstar) & valid
      pos2 = p_nx + plsc.cumsum(jnp.where(meq, ones, zeros)) - 1
      plsc.store_scatter(dst_ku, [pos2], ku, mask=meq)
      plsc.store_scatter(dst_idx, [pos2], gidx, mask=meq)
      return (p_gt + plsc.all_reduce_population_count(mgt),
              p_nx + plsc.all_reduce_population_count(meq))
    ptr_gt, n_cur = lax.fori_loop(0, nv, ploop, (ptr_gt, zeros))

  ptr_eq = n_cur
  eq_idx = curb_idx

  val16[...] = jnp.where(iota == 0, ptr_gt, jnp.where(iota == 1, ptr_eq, 0))
  idx16[...] = jnp.where(iota == 0, sid,
                         jnp.where(iota == 1, NS + sid, 2 * NS + iota))
  pltpu.sync_copy(val16, sh_cnts2.at[idx16])
  plsc.subcore_barrier()
  pltpu.sync_copy(sh_cnts2.at[pl.ds(0, 2 * NS)], cnt2_l)
  gtc = cnt2_l[pl.ds(0, L)]
  neqc = cnt2_l[pl.ds(L, L)]
  sid_v = jnp.full((L,), sid, jnp.int32)
  cg = plsc.cumsum(gtc)
  tmp16[...] = cg - gtc
  gt_base = plsc.load_gather(tmp16, [sid_v])
  tmp16[...] = cg
  ngt_tot = plsc.load_gather(tmp16, [jnp.full((L,), 15, jnp.int32)])
  tmp16[...] = plsc.cumsum(neqc) - neqc
  eq_base = plsc.load_gather(tmp16, [sid_v])
  take = jnp.clip(jnp.int32(K_OUT) - ngt_tot - eq_base, 0, ptr_eq)

  nev = (jnp.max(ptr_gt) + L - 1) // L

  def sort_one(t, _):
    i = t * L + iota
    ke = gt_ku[pl.ds(t * L, L)]
    ie = gt_idx[pl.ds(t * L, L)]

    def sinner(j, acc):
      jv = zeros + j
      kj = plsc.load_gather(gt_ku, [jv])
      ij = plsc.load_gather(gt_idx, [jv])
      c = (kj > ke) | ((kj == ke) & (ij < ie))
      return acc + jnp.where(c, ones, zeros)
    r = lax.fori_loop(0, jnp.max(ptr_gt), sinner, zeros)
    mvalid = i < ptr_gt
    plsc.store_scatter(srt_ku, [r], ke, mask=mvalid)
    plsc.store_scatter(srt_idx, [r], ie, mask=mvalid)
    return 0
  lax.fori_loop(0, nev, sort_one, 0)

  dump = KPAD + sid * L + iota

  def scat_gt(t, _):
    i = t * L + iota
    idx16[...] = jnp.where(i < ptr_gt, gt_base + i, dump)
    pltpu.sync_copy(srt_ku.at[pl.ds(t * L, L)], sh_selku.at[idx16])
    pltpu.sync_copy(srt_idx.at[pl.ds(t * L, L)], sh_selidx.at[idx16])
    return 0
  lax.fori_loop(0, nev, scat_gt, 0)

  nch2 = (jnp.max(take) + L - 1) // L

  def scat_eq(t, _):
    i = t * L + iota
    idx16[...] = jnp.where(i < take, ngt_tot + eq_base + i, dump)
    pltpu.sync_copy(eq_idx.at[pl.ds(t * L, L)], sh_outidx.at[idx16])
    return 0
  lax.fori_loop(0, nch2, scat_eq, 0)
  plsc.subcore_barrier()

  pltpu.sync_copy(sh_selku.at[pl.ds(0, KPAD)], loc_ku)
  pltpu.sync_copy(sh_selidx.at[pl.ds(0, KPAD)], loc_idx)
  stage16[...] = cg - gtc
  val16[...] = gtc
  maxv = zeros + jnp.max(gtc)
  nsteps = jnp.max(
      plsc.all_reduce_population_count((ones << iota) <= maxv))

  def search_one(ev, _):
    ke = srt_ku[pl.ds(ev * L, L)]
    ie = srt_idx[pl.ds(ev * L, L)]

    def tloop(t, rank):
      tv = zeros + t
      bt = plsc.load_gather(stage16, [tv])
      ct = plsc.load_gather(val16, [tv])

      def bstep(s, lohi):
        lo, hi = lohi
        act = lo < hi
        mid = (lo + hi) >> 1
        addr = bt + jnp.minimum(mid, jnp.maximum(ct - 1, zeros))
        kx = plsc.load_gather(loc_ku, [addr])
        ix = plsc.load_gather(loc_idx, [addr])
        g = (kx > ke) | ((kx == ke) & (ix < ie))
        return (jnp.where(act & g, mid + 1, lo),
                jnp.where(act & (~g), mid, hi))
      lo, _hi = lax.fori_loop(0, nsteps, bstep, (zeros, ct))
      return rank + lo
    rank = lax.fori_loop(0, NS, tloop, zeros)
    rnk_v[pl.ds(ev * L, L)] = rank
    return 0
  lax.fori_loop(0, nev, search_one, 0)

  def scat_rank(t, _):
    i = t * L + iota
    idx16[...] = jnp.where(i < ptr_gt, rnk_v[pl.ds(t * L, L)], dump)
    pltpu.sync_copy(srt_idx.at[pl.ds(t * L, L)], sh_outidx.at[idx16])
    return 0
  lax.fori_loop(0, nev, scat_rank, 0)
  plsc.subcore_barrier()

  wid = cid * NS + sid
  GL = K_OUT - (NS * NC - 1) * G

  @pl.when(wid < NS * NC - 1)
  def _():
    pltpu.sync_copy(sh_outidx.at[pl.ds(wid * G, G)], outidx_v)
    for q in range(G // L):
      v = outidx_v[pl.ds(q * L, L)]
      outidx_v[pl.ds(q * L, L)] = jnp.clip(v, 0, N_ROWS - 1)
    pltpu.async_copy(x_hbm.at[outidx_v], rows_v, sem).wait()
    pltpu.sync_copy(rows_v, out_hbm.at[pl.ds(wid * G, G)])

  @pl.when(wid == NS * NC - 1)
  def _():
    pltpu.sync_copy(sh_outidx.at[pl.ds(wid * G, GL)],
                    outidx_v.at[pl.ds(0, GL)])
    for q in range(GL // L):
      v = outidx_v[pl.ds(q * L, L)]
      outidx_v[pl.ds(q * L, L)] = jnp.clip(v, 0, N_ROWS - 1)
    pltpu.async_copy(x_hbm.at[outidx_v.at[pl.ds(0, GL)]],
                     rows_v.at[pl.ds(0, GL)], sem).wait()
    pltpu.sync_copy(rows_v.at[pl.ds(0, GL)], out_hbm.at[pl.ds(wid * G, GL)])


_sortpool = pl.kernel(
    _body,
    out_type=jax.ShapeDtypeStruct((K_OUT, N_FEAT), jnp.float32),
    mesh=plsc.VectorSubcoreMesh(core_axis_name="c", subcore_axis_name="s"),
    compiler_params=pltpu.CompilerParams(needs_layout_passes=False),
    scratch_types=[
        pltpu.VMEM((C,), jnp.float32),
        pltpu.VMEM((KPAD,), jnp.int32),
        pltpu.VMEM((KPAD,), jnp.int32),
        pltpu.VMEM((C,), jnp.int32),
        pltpu.VMEM((C,), jnp.int32),
        pltpu.VMEM((C,), jnp.int32),
        pltpu.VMEM((C,), jnp.int32),
        pltpu.VMEM((KPAD,), jnp.int32),
        pltpu.VMEM((KPAD,), jnp.int32),
        pltpu.VMEM((KPAD,), jnp.int32),
        pltpu.VMEM((KPAD,), jnp.int32),
        pltpu.VMEM((KPAD,), jnp.int32),
        pltpu.VMEM((G,), jnp.int32),
        pltpu.VMEM((G, N_FEAT), jnp.float32),
        pltpu.VMEM((NS, L), jnp.int32),
        pltpu.VMEM((2 * NS,), jnp.int32),
        pltpu.VMEM((L,), jnp.int32),
        pltpu.VMEM((L,), jnp.int32),
        pltpu.VMEM((L,), jnp.int32),
        pltpu.VMEM((L,), jnp.int32),
        pltpu.VMEM((16 * L,), jnp.int32),
        pltpu.VMEM_SHARED((2, NS, L), jnp.int32),
        pltpu.VMEM_SHARED((3 * NS,), jnp.int32),
        pltpu.VMEM_SHARED((SEL,), jnp.int32),
        pltpu.VMEM_SHARED((SEL,), jnp.int32),
        pltpu.VMEM_SHARED((SEL,), jnp.int32),
        pltpu.SemaphoreType.DMA,
    ],
)


def kernel(X):
  return _sortpool(X, X[:, N_FEAT - 1])
